# Initial kernel scaffold; baseline (speedup 1.0000x reference)
import jax
import jax.numpy as jnp
from jax.experimental import pallas as pl
from jax.experimental.pallas import tpu as pltpu

B, S, H, D = 1, 2048, 12, 64
DMODEL = H * D
DFF = 2048
E = 8
BF = 512
NJ = DFF // BF
T = B * S


def _moe_kernel(x_ref, gw_ref, w1_ref, w3_ref, w2_ref, out_ref,
                xb_ref, am1_ref, am2_ref, wv1_ref, wv2_ref):
    e = pl.program_id(0)
    j = pl.program_id(1)

    @pl.when((e == 0) & (j == 0))
    def _init():
        x = x_ref[...]
        xb_ref[...] = x.astype(jnp.bfloat16)
        logits = jnp.dot(x, gw_ref[...], preferred_element_type=jnp.float32)
        am1 = jnp.argmax(logits, axis=1)[:, None]
        eids = jax.lax.broadcasted_iota(jnp.int32, logits.shape, 1)
        m1 = jnp.max(logits, axis=1, keepdims=True)
        masked = jnp.where(eids == am1, -jnp.inf, logits)
        am2 = jnp.argmax(masked, axis=1)[:, None]
        m2 = jnp.max(masked, axis=1, keepdims=True)
        z = jnp.exp(m2 - m1)
        w1v = 1.0 / (1.0 + z)
        am1_ref[...] = am1.astype(jnp.int32)
        am2_ref[...] = am2.astype(jnp.int32)
        wv1_ref[...] = w1v
        wv2_ref[...] = 1.0 - w1v
        out_ref[...] = jnp.zeros_like(out_ref)

    xb = xb_ref[...]
    g = jnp.dot(xb, w1_ref[0].astype(jnp.bfloat16),
                preferred_element_type=jnp.float32)
    u = jnp.dot(xb, w3_ref[0].astype(jnp.bfloat16),
                preferred_element_type=jnp.float32)
    g = g * jax.nn.sigmoid(g)
    h = (g * u).astype(jnp.bfloat16)
    part = jnp.dot(h, w2_ref[0].astype(jnp.bfloat16),
                   preferred_element_type=jnp.float32)
    w_e = (jnp.where(am1_ref[...] == e, wv1_ref[...], 0.0)
           + jnp.where(am2_ref[...] == e, wv2_ref[...], 0.0))
    out_ref[...] += w_e * part


@jax.jit
def _moe(x, gate_W, W1, W2, W3):
    grid = (E, NJ)
    return pl.pallas_call(
        _moe_kernel,
        grid=grid,
        in_specs=[
            pl.BlockSpec((T, DMODEL), lambda e, j: (0, 0)),
            pl.BlockSpec((DMODEL, E), lambda e, j: (0, 0)),
            pl.BlockSpec((1, DMODEL, BF), lambda e, j: (e, 0, j)),
            pl.BlockSpec((1, BF, DMODEL), lambda e, j: (e, j, 0)),
            pl.BlockSpec((1, DMODEL, BF), lambda e, j: (e, 0, j)),
        ],
        out_specs=pl.BlockSpec((T, DMODEL), lambda e, j: (0, 0)),
        out_shape=jax.ShapeDtypeStruct((T, DMODEL), jnp.float32),
        scratch_shapes=[
            pltpu.VMEM((T, DMODEL), jnp.bfloat16),
            pltpu.VMEM((T, 1), jnp.int32),
            pltpu.VMEM((T, 1), jnp.int32),
            pltpu.VMEM((T, 1), jnp.float32),
            pltpu.VMEM((T, 1), jnp.float32),
        ],
        compiler_params=pltpu.CompilerParams(
            dimension_semantics=("arbitrary", "arbitrary"),
        ),
    )(x, gate_W, W1, W3, W2)


def kernel(stm, gate_W, W1, W2, W3):
    b, s, h, dh = stm.shape
    x = stm.reshape(b * s, h * dh)
    out = _moe(x, gate_W, W1, W2, W3)
    return out.reshape(b, s, h, dh)


# speedup vs baseline: 3.1173x; 3.1173x over previous
import jax
import jax.numpy as jnp
from jax.experimental import pallas as pl
from jax.experimental.pallas import tpu as pltpu

B, S, H, D = 1, 2048, 12, 64
DMODEL = H * D
DFF = 2048
E = 8
BF = 512
NJ = DFF // BF
T = B * S


def _moe_kernel(x_ref, gw_ref, w1_ref, w3_ref, w2_ref, out_ref,
                xb_ref, am1_ref, am2_ref, wv1_ref, wv2_ref):
    e = pl.program_id(0)
    j = pl.program_id(1)

    @pl.when((e == 0) & (j == 0))
    def _init():
        x = x_ref[...]
        xb_ref[...] = x.astype(jnp.bfloat16)
        logits = jnp.dot(x, gw_ref[...], preferred_element_type=jnp.float32)
        am1 = jnp.argmax(logits, axis=1)[:, None]
        eids = jax.lax.broadcasted_iota(jnp.int32, logits.shape, 1)
        m1 = jnp.max(logits, axis=1, keepdims=True)
        masked = jnp.where(eids == am1, -jnp.inf, logits)
        am2 = jnp.argmax(masked, axis=1)[:, None]
        m2 = jnp.max(masked, axis=1, keepdims=True)
        z = jnp.exp(m2 - m1)
        w1v = 1.0 / (1.0 + z)
        am1_ref[...] = am1.astype(jnp.int32)
        am2_ref[...] = am2.astype(jnp.int32)
        wv1_ref[...] = w1v
        wv2_ref[...] = 1.0 - w1v
        out_ref[...] = jnp.zeros_like(out_ref)

    xb = xb_ref[...]
    g = jnp.dot(xb, w1_ref[0].astype(jnp.bfloat16),
                preferred_element_type=jnp.float32)
    u = jnp.dot(xb, w3_ref[0].astype(jnp.bfloat16),
                preferred_element_type=jnp.float32)
    g = g * jax.nn.sigmoid(g)
    h = (g * u).astype(jnp.bfloat16)
    part = jnp.dot(h, w2_ref[0].astype(jnp.bfloat16),
                   preferred_element_type=jnp.float32)
    w_e = (jnp.where(am1_ref[...] == e, wv1_ref[...], 0.0)
           + jnp.where(am2_ref[...] == e, wv2_ref[...], 0.0))
    out_ref[...] += w_e * part


@jax.jit
def _moe(x, gate_W, W1, W2, W3):
    grid = (E, NJ)
    return pl.pallas_call(
        _moe_kernel,
        grid=grid,
        in_specs=[
            pl.BlockSpec((T, DMODEL), lambda e, j: (0, 0)),
            pl.BlockSpec((DMODEL, E), lambda e, j: (0, 0)),
            pl.BlockSpec((1, DMODEL, BF), lambda e, j: (e, 0, j)),
            pl.BlockSpec((1, DMODEL, BF), lambda e, j: (e, 0, j)),
            pl.BlockSpec((1, BF, DMODEL), lambda e, j: (e, j, 0)),
        ],
        out_specs=pl.BlockSpec((T, DMODEL), lambda e, j: (0, 0)),
        out_shape=jax.ShapeDtypeStruct((T, DMODEL), jnp.float32),
        scratch_shapes=[
            pltpu.VMEM((T, DMODEL), jnp.bfloat16),
            pltpu.VMEM((T, 1), jnp.int32),
            pltpu.VMEM((T, 1), jnp.int32),
            pltpu.VMEM((T, 1), jnp.float32),
            pltpu.VMEM((T, 1), jnp.float32),
        ],
        compiler_params=pltpu.CompilerParams(
            dimension_semantics=("arbitrary", "arbitrary"),
        ),
    )(x, gate_W, W1, W3, W2)


def kernel(stm, gate_W, W1, W2, W3):
    b, s, h, dh = stm.shape
    x = stm.reshape(b * s, h * dh)
    out = _moe(x, gate_W, W1, W2, W3)
    return out.reshape(b, s, h, dh)
